# Initial kernel scaffold; baseline (speedup 1.0000x reference)
import jax
import jax.numpy as jnp
from jax.experimental import pallas as pl


def kernel(q, rel_pos_table, W_ur, W_i, scale, seq_len):
    raise NotImplementedError("write your pallas kernel here")


# speedup vs baseline: 36.2896x; 36.2896x over previous
import functools
import math

import numpy as np
import jax
import jax.numpy as jnp
from jax import lax
from jax.experimental import pallas as pl
from jax.experimental.pallas import tpu as pltpu
from jax.experimental.pallas import tpu_sc as plsc

NUM_HEADS = 16
HEAD_DIM = 64
NUM_BUCKETS = 320
MAX_DISTANCE = 800
T = 2048
VLEN = 2 * T + 16
VROW = 2 * T
NSHIFT = 8
L = 16

QROWS = NUM_HEADS * T
QCHUNK = 128
ROWS_PER_WORKER = T // 2
GROUP = 8


def _bucket_vector() -> np.ndarray:
    d = np.arange(VLEN, dtype=np.int64) - (T - 1)
    half = NUM_BUCKETS // 2
    threshold = half // 2
    sign = (d >= 0).astype(np.int64)
    a = np.abs(d)
    log_ratio = np.log(np.clip(a.astype(np.float32), 1.0, None) / np.float32(threshold)) \
        / np.float32(math.log(MAX_DISTANCE / threshold))
    log_pos = np.minimum(
        (np.float32(threshold) + log_ratio * np.float32(half - threshold)).astype(np.int64),
        half - 1)
    b = np.where(a < threshold, a, log_pos) + sign * half
    b = np.clip(b, 0, NUM_BUCKETS - 1).astype(np.int32)
    b[2 * T - 1:] = b[2 * T - 2]
    return b


_BUCKETS = _bucket_vector()


def _sc_body(q_hbm, tab_hbm, w_hbm, scale_hbm, b_hbm, out_hbm,
             qbuf, tabv, wbuf, svec, bvec, vbuf, pvec, shared_part, dsem):
    c = lax.axis_index("c")
    s = lax.axis_index("s")
    h = s

    pltpu.sync_copy(tab_hbm, tabv)
    pltpu.sync_copy(b_hbm, bvec)
    pltpu.sync_copy(w_hbm, wbuf)
    pltpu.sync_copy(scale_hbm, svec)

    nj = HEAD_DIM // L

    def row_body(r, accs):
        base = r * HEAD_DIM
        return tuple(accs[j] + qbuf[pl.ds(base + j * L, L)] for j in range(nj))

    accs = tuple(jnp.zeros((L,), jnp.float32) for _ in range(nj))
    for chunk in range(T // QCHUNK):
        pltpu.sync_copy(
            q_hbm.at[pl.ds((s * T + chunk * QCHUNK) * HEAD_DIM, QCHUNK * HEAD_DIM)],
            qbuf)
        accs = lax.fori_loop(0, QCHUNK, row_body, accs)

    for j in range(nj):
        pvec[pl.ds(j * L, L)] = accs[j]
    pltpu.sync_copy(pvec, shared_part.at[pl.ds(s * HEAD_DIM, HEAD_DIM)])
    plsc.subcore_barrier()
    pltpu.sync_copy(shared_part, qbuf.at[pl.ds(0, NUM_HEADS * HEAD_DIM)])

    gm = []
    for j in range(nj):
        t = qbuf[pl.ds(j * L, L)]
        for i in range(1, NUM_HEADS):
            t = t + qbuf[pl.ds(i * HEAD_DIM + j * L, L)]
        gm.append(t * (1.0 / QROWS))

    lanes = jnp.arange(L, dtype=jnp.int32)

    def lane_sum_splat(vec):
        for stride in (8, 4, 2, 1):
            pvec[pl.ds(0, L)] = vec
            vec = vec + plsc.load_gather(pvec, [lanes ^ stride])
        return vec

    su = jnp.zeros((L,), jnp.float32)
    si = jnp.zeros((L,), jnp.float32)
    for j in range(nj):
        su = su + gm[j] * wbuf[pl.ds(h * HEAD_DIM + j * L, L)]
        si = si + gm[j] * wbuf[pl.ds((h + NUM_HEADS) * HEAD_DIM + j * L, L)]
    su = lane_sum_splat(su)
    si = lane_sum_splat(si)
    gr = 1.0 / (1.0 + jnp.exp(-su))
    gu = 1.0 / (1.0 + jnp.exp(-si))
    scale_h = plsc.load_gather(svec, [jnp.full((L,), h, jnp.int32)])
    one_plus_g = 1.0 + gu * scale_h * gr

    def v_body(i, _):
        m = i * L
        for sft in range(NSHIFT):
            bidx = plsc.load_gather(bvec, [lanes + (m + sft)])
            vals = plsc.load_gather(tabv, [bidx * NUM_HEADS + h])
            vbuf[pl.ds(sft * VROW + m, L)] = vals * one_plus_g
        return 0

    lax.fori_loop(0, VROW // L, v_body, 0)

    row0 = c * ROWS_PER_WORKER

    def group_body(gidx, _):
        rb = row0 + gidx * GROUP
        o8 = (T - NSHIFT) - rb
        copies = [
            pltpu.async_copy(
                vbuf.at[pl.ds((NSHIFT - 1 - k) * VROW + o8, T)],
                out_hbm.at[pl.ds((h * T + rb + k) * T, T)],
                dsem)
            for k in range(GROUP)
        ]
        for cp in copies:
            cp.wait()
        return 0

    lax.fori_loop(0, ROWS_PER_WORKER // GROUP, group_body, 0)


@jax.jit
def _run(q1d, tabf, wcat, scale, bconst):
    mesh = plsc.VectorSubcoreMesh(core_axis_name="c", subcore_axis_name="s")
    kfn = functools.partial(
        pl.kernel,
        mesh=mesh,
        compiler_params=pltpu.CompilerParams(needs_layout_passes=False),
        out_type=jax.ShapeDtypeStruct((NUM_HEADS * T * T,), jnp.float32),
        scratch_types=[
            pltpu.VMEM((QCHUNK * HEAD_DIM,), jnp.float32),
            pltpu.VMEM((NUM_BUCKETS * NUM_HEADS,), jnp.float32),
            pltpu.VMEM((2 * NUM_HEADS * HEAD_DIM,), jnp.float32),
            pltpu.VMEM((NUM_HEADS,), jnp.float32),
            pltpu.VMEM((VLEN,), jnp.int32),
            pltpu.VMEM((NSHIFT * VROW,), jnp.float32),
            pltpu.VMEM((HEAD_DIM,), jnp.float32),
            pltpu.VMEM_SHARED((NUM_HEADS * HEAD_DIM,), jnp.float32),
            pltpu.SemaphoreType.DMA,
        ],
    )(_sc_body)
    return kfn(q1d, tabf, wcat, scale, bconst)


def kernel(q, rel_pos_table, W_ur, W_i, scale, seq_len):
    B, H, Tq, D = q.shape
    q1d = q.reshape(-1)
    tabf = rel_pos_table.reshape(-1)
    wcat = jnp.concatenate([W_ur.reshape(-1), W_i.reshape(-1)])
    bconst = jnp.asarray(_BUCKETS)
    out = _run(q1d, tabf, wcat, scale, bconst)
    return out.reshape(B, H, Tq, Tq)


# speedup vs baseline: 36.3498x; 1.0017x over previous
import functools
import math

import numpy as np
import jax
import jax.numpy as jnp
from jax import lax
from jax.experimental import pallas as pl
from jax.experimental.pallas import tpu as pltpu
from jax.experimental.pallas import tpu_sc as plsc

NUM_HEADS = 16
HEAD_DIM = 64
NUM_BUCKETS = 320
MAX_DISTANCE = 800
T = 2048
VLEN = 2 * T + 16
VROW = 2 * T
NSHIFT = 8
L = 16

QROWS = NUM_HEADS * T
QCHUNK = 128
ROWS_PER_WORKER = T // 2
GROUP = 8


def _bucket_vector() -> np.ndarray:
    d = np.arange(VLEN, dtype=np.int64) - (T - 1)
    half = NUM_BUCKETS // 2
    threshold = half // 2
    sign = (d >= 0).astype(np.int64)
    a = np.abs(d)
    log_ratio = np.log(np.clip(a.astype(np.float32), 1.0, None) / np.float32(threshold)) \
        / np.float32(math.log(MAX_DISTANCE / threshold))
    log_pos = np.minimum(
        (np.float32(threshold) + log_ratio * np.float32(half - threshold)).astype(np.int64),
        half - 1)
    b = np.where(a < threshold, a, log_pos) + sign * half
    b = np.clip(b, 0, NUM_BUCKETS - 1).astype(np.int32)
    b[2 * T - 1:] = b[2 * T - 2]
    return b


_BUCKETS = _bucket_vector()


def _sc_body(q_hbm, tab_hbm, w_hbm, scale_hbm, b_hbm, out_hbm,
             qbuf, tabv, wbuf, svec, bvec, vbuf, pvec, shared_part, dsem):
    c = lax.axis_index("c")
    s = lax.axis_index("s")
    h = s

    pltpu.sync_copy(tab_hbm, tabv)
    pltpu.sync_copy(b_hbm, bvec)
    pltpu.sync_copy(w_hbm, wbuf)
    pltpu.sync_copy(scale_hbm, svec)

    nj = HEAD_DIM // L

    def row_body(r, accs):
        base = r * HEAD_DIM
        return tuple(accs[j] + qbuf[pl.ds(base + j * L, L)] for j in range(nj))

    accs = tuple(jnp.zeros((L,), jnp.float32) for _ in range(nj))
    for chunk in range(T // QCHUNK):
        pltpu.sync_copy(
            q_hbm.at[pl.ds((s * T + chunk * QCHUNK) * HEAD_DIM, QCHUNK * HEAD_DIM)],
            qbuf)
        accs = lax.fori_loop(0, QCHUNK, row_body, accs)

    for j in range(nj):
        pvec[pl.ds(j * L, L)] = accs[j]
    pltpu.sync_copy(pvec, shared_part.at[pl.ds(s * HEAD_DIM, HEAD_DIM)])
    plsc.subcore_barrier()
    pltpu.sync_copy(shared_part, qbuf.at[pl.ds(0, NUM_HEADS * HEAD_DIM)])

    gm = []
    for j in range(nj):
        t = qbuf[pl.ds(j * L, L)]
        for i in range(1, NUM_HEADS):
            t = t + qbuf[pl.ds(i * HEAD_DIM + j * L, L)]
        gm.append(t * (1.0 / QROWS))

    lanes = jnp.arange(L, dtype=jnp.int32)

    def lane_sum_splat(vec):
        for stride in (8, 4, 2, 1):
            pvec[pl.ds(0, L)] = vec
            vec = vec + plsc.load_gather(pvec, [lanes ^ stride])
        return vec

    su = jnp.zeros((L,), jnp.float32)
    si = jnp.zeros((L,), jnp.float32)
    for j in range(nj):
        su = su + gm[j] * wbuf[pl.ds(h * HEAD_DIM + j * L, L)]
        si = si + gm[j] * wbuf[pl.ds((h + NUM_HEADS) * HEAD_DIM + j * L, L)]
    su = lane_sum_splat(su)
    si = lane_sum_splat(si)
    gr = 1.0 / (1.0 + jnp.exp(-su))
    gu = 1.0 / (1.0 + jnp.exp(-si))
    scale_h = plsc.load_gather(svec, [jnp.full((L,), h, jnp.int32)])
    one_plus_g = 1.0 + gu * scale_h * gr

    def v_body(i, _):
        m = i * L
        for k in range(NSHIFT):
            sft = NSHIFT - 1 - k
            bidx = plsc.load_gather(bvec, [lanes + (m + sft)])
            vals = plsc.load_gather(tabv, [bidx * NUM_HEADS + h])
            vbuf[k, pl.ds(m, L)] = vals * one_plus_g
        return 0

    lax.fori_loop(0, VROW // L, v_body, 0)

    row0 = c * ROWS_PER_WORKER

    def group_body(gidx, _):
        copies = []
        for k in range(GROUP):
            rb = row0 + (gidx * GROUP + k) * NSHIFT
            o8 = (T - NSHIFT) - rb
            copies.append(pltpu.async_copy(
                vbuf.at[:, pl.ds(o8, T)],
                out_hbm.at[pl.ds(h * T + rb, NSHIFT)],
                dsem))
        for cp in copies:
            cp.wait()
        return 0

    lax.fori_loop(0, ROWS_PER_WORKER // (GROUP * NSHIFT), group_body, 0)


@jax.jit
def _run(q1d, tabf, wcat, scale, bconst):
    mesh = plsc.VectorSubcoreMesh(core_axis_name="c", subcore_axis_name="s")
    kfn = functools.partial(
        pl.kernel,
        mesh=mesh,
        compiler_params=pltpu.CompilerParams(
            needs_layout_passes=False, use_tc_tiling_on_sc=False),
        out_type=jax.ShapeDtypeStruct((NUM_HEADS * T, T), jnp.float32),
        scratch_types=[
            pltpu.VMEM((QCHUNK * HEAD_DIM,), jnp.float32),
            pltpu.VMEM((NUM_BUCKETS * NUM_HEADS,), jnp.float32),
            pltpu.VMEM((2 * NUM_HEADS * HEAD_DIM,), jnp.float32),
            pltpu.VMEM((NUM_HEADS,), jnp.float32),
            pltpu.VMEM((VLEN,), jnp.int32),
            pltpu.VMEM((NSHIFT, VROW), jnp.float32),
            pltpu.VMEM((HEAD_DIM,), jnp.float32),
            pltpu.VMEM_SHARED((NUM_HEADS * HEAD_DIM,), jnp.float32),
            pltpu.SemaphoreType.DMA,
        ],
    )(_sc_body)
    return kfn(q1d, tabf, wcat, scale, bconst)


def kernel(q, rel_pos_table, W_ur, W_i, scale, seq_len):
    B, H, Tq, D = q.shape
    q1d = q.reshape(-1)
    tabf = rel_pos_table.reshape(-1)
    wcat = jnp.concatenate([W_ur.reshape(-1), W_i.reshape(-1)])
    bconst = jnp.asarray(_BUCKETS)
    out = _run(q1d, tabf, wcat, scale, bconst)
    return out.reshape(B, H, Tq, Tq)
